# baseline (speedup 1.0000x reference)
import functools

import jax
import jax.numpy as jnp
from jax import lax
from jax.experimental import pallas as pl
from jax.experimental.pallas import tpu as pltpu
from jax.experimental.pallas import tpu_sc as plsc

N_N = 100000
CH_N = 8
CH_E = 8
HID = 16
MAX_VALUE = 1000000.0

NC = 2
NS = 16
NW = NC * NS
G = 40
BLK = G * 128


def _soft_clamp(x):
    return MAX_VALUE * jnp.tanh(x / MAX_VALUE)


@functools.lru_cache(maxsize=None)
def _make_gather(E):
    nblocks = E // BLK
    mesh = plsc.VectorSubcoreMesh(core_axis_name="c", subcore_axis_name="s")

    @functools.partial(
        pl.kernel,
        mesh=mesh,
        compiler_params=pltpu.CompilerParams(use_tc_tiling_on_sc=False),
        out_type=[jax.ShapeDtypeStruct((E // 128, 128, CH_N), jnp.float32),
                  jax.ShapeDtypeStruct((E // 128, 128, CH_N), jnp.float32)],
        scratch_types=[pltpu.VMEM((G, 128), jnp.int32),
                       pltpu.VMEM((G, 128), jnp.int32),
                       pltpu.VMEM((G, 128, CH_N), jnp.float32),
                       pltpu.VMEM((G, 128, CH_N), jnp.float32),
                       pltpu.SemaphoreType.DMA,
                       pltpu.SemaphoreType.DMA],
    )
    def gather_k(nodes_hbm, src_hbm, dst_hbm, outa_hbm, outb_hbm,
                 ia_v, ib_v, ra_v, rb_v, sema, semb):
        wid = lax.axis_index("s") * NC + lax.axis_index("c")
        nfull = nblocks // NW
        extra = nblocks - nfull * NW
        count = nfull + jnp.where(wid < extra, 1, 0)

        def body(i, carry):
            b = wid + i * NW
            pltpu.sync_copy(src_hbm.at[pl.ds(b * G, G)], ia_v)
            pltpu.sync_copy(dst_hbm.at[pl.ds(b * G, G)], ib_v)
            copies = []
            for j in range(G):
                copies.append(pltpu.async_copy(
                    nodes_hbm.at[ia_v.at[j]], ra_v.at[j], sema))
                copies.append(pltpu.async_copy(
                    nodes_hbm.at[ib_v.at[j]], rb_v.at[j], semb))
            for c in copies:
                c.wait()
            pltpu.sync_copy(ra_v, outa_hbm.at[pl.ds(b * G, G)])
            pltpu.sync_copy(rb_v, outb_hbm.at[pl.ds(b * G, G)])
            return carry

        lax.fori_loop(0, count, body, 0)

    return gather_k


@functools.lru_cache(maxsize=None)
def _make_scatter(E):
    nblocks = E // BLK
    share = (N_N // NS) // 8 * 8
    last = N_N - (NS - 1) * share
    mesh = plsc.VectorSubcoreMesh(core_axis_name="c", subcore_axis_name="s")

    @functools.partial(
        pl.kernel,
        mesh=mesh,
        compiler_params=pltpu.CompilerParams(use_tc_tiling_on_sc=False),
        out_type=[jax.ShapeDtypeStruct((N_N, CH_N), jnp.float32),
                  jax.ShapeDtypeStruct((N_N, CH_N), jnp.float32)],
        scratch_types=[pltpu.VMEM((G, 128), jnp.int32),
                       pltpu.VMEM((G, 128, CH_N), jnp.float32),
                       pltpu.VMEM_SHARED((N_N, CH_N), jnp.float32),
                       pltpu.SemaphoreType.DMA],
    )
    def scatter_k(src_hbm, dst_hbm, ma_hbm, mb_hbm, zeros_hbm,
                  outa_hbm, outb_hbm, idx_v, upd_v, agg_sh, sem):
        cid = lax.axis_index("c")
        sid = lax.axis_index("s")

        @pl.when(sid < NS - 1)
        def _():
            pltpu.sync_copy(zeros_hbm.at[pl.ds(sid * share, share)],
                            agg_sh.at[pl.ds(sid * share, share)])

        @pl.when(sid == NS - 1)
        def _():
            pltpu.sync_copy(zeros_hbm.at[pl.ds((NS - 1) * share, last)],
                            agg_sh.at[pl.ds((NS - 1) * share, last)])

        plsc.subcore_barrier()

        nfull = nblocks // NS
        extra = nblocks - nfull * NS
        count = nfull + jnp.where(sid < extra, 1, 0)

        def body(i, carry):
            b = sid + i * NS

            @pl.when(cid == 0)
            def _():
                pltpu.sync_copy(src_hbm.at[pl.ds(b * G, G)], idx_v)
                pltpu.sync_copy(ma_hbm.at[pl.ds(b * G, G)], upd_v)

            @pl.when(cid == 1)
            def _():
                pltpu.sync_copy(dst_hbm.at[pl.ds(b * G, G)], idx_v)
                pltpu.sync_copy(mb_hbm.at[pl.ds(b * G, G)], upd_v)

            copies = []
            for j in range(G):
                copies.append(pltpu.async_copy(
                    upd_v.at[j], agg_sh.at[idx_v.at[j]], sem, add=True))
            for c in copies:
                c.wait()
            return carry

        lax.fori_loop(0, count, body, 0)
        plsc.subcore_barrier()

        @pl.when((cid == 0) & (sid < NS - 1))
        def _():
            pltpu.sync_copy(agg_sh.at[pl.ds(sid * share, share)],
                            outa_hbm.at[pl.ds(sid * share, share)])

        @pl.when((cid == 0) & (sid == NS - 1))
        def _():
            pltpu.sync_copy(agg_sh.at[pl.ds((NS - 1) * share, last)],
                            outa_hbm.at[pl.ds((NS - 1) * share, last)])

        @pl.when((cid == 1) & (sid < NS - 1))
        def _():
            pltpu.sync_copy(agg_sh.at[pl.ds(sid * share, share)],
                            outb_hbm.at[pl.ds(sid * share, share)])

        @pl.when((cid == 1) & (sid == NS - 1))
        def _():
            pltpu.sync_copy(agg_sh.at[pl.ds((NS - 1) * share, last)],
                            outb_hbm.at[pl.ds((NS - 1) * share, last)])

    return scatter_k


def _edge_body(na_ref, nb_ref, eg_ref, a1_ref, b1_ref, e1_ref, bias1_ref,
               w2a_ref, w2b_ref, w2e_ref, b2a_ref, b2b_ref, b2e_ref,
               ma_ref, mb_ref, ne_ref):
    eg = eg_ref[...]
    h = jnp.tanh(na_ref[...] @ a1_ref[...] + nb_ref[...] @ b1_ref[...]
                 + eg @ e1_ref[...] + bias1_ref[...])
    ma_ref[...] = h @ w2a_ref[...] + b2a_ref[...]
    mb_ref[...] = h @ w2b_ref[...] + b2b_ref[...]
    ne_ref[...] = _soft_clamp(eg + h @ w2e_ref[...] + b2e_ref[...])


@functools.lru_cache(maxsize=None)
def _make_edge_mlp(rows):
    R = 1000
    grid = rows // R
    full = lambda shape: pl.BlockSpec(shape, lambda i: (0, 0))
    blk = pl.BlockSpec((R, 128), lambda i: (i, 0))
    return pl.pallas_call(
        _edge_body,
        grid=(grid,),
        in_specs=[blk, blk, blk,
                  full((128, 256)), full((128, 256)), full((128, 256)),
                  full((1, 256)),
                  full((256, 128)), full((256, 128)), full((256, 128)),
                  full((1, 128)), full((1, 128)), full((1, 128))],
        out_specs=[blk, blk, blk],
        out_shape=[jax.ShapeDtypeStruct((rows, 128), jnp.float32)] * 3,
    )


def _node_body(nd_ref, aa_ref, ab_ref, u1n_ref, u1a_ref, u1b_ref, ub1_ref,
               u2_ref, ub2_ref, out_ref):
    nd = nd_ref[...]
    hu = jnp.tanh(nd @ u1n_ref[...] + aa_ref[...] @ u1a_ref[...]
                  + ab_ref[...] @ u1b_ref[...] + ub1_ref[...])
    out_ref[...] = _soft_clamp(nd + hu @ u2_ref[...] + ub2_ref[...])


@functools.lru_cache(maxsize=None)
def _make_node_mlp(rows):
    return pl.pallas_call(
        _node_body,
        out_shape=jax.ShapeDtypeStruct((rows, 128), jnp.float32),
    )


def kernel(nodes, edges, edge_index, msg_W1, msg_b1, msg_W2, msg_b2,
           upd_W1, upd_b1, upd_W2, upd_b2):
    E = edges.shape[0]
    f32 = jnp.float32
    src = edge_index[0].astype(jnp.int32).reshape(E // 128, 128)
    dst = edge_index[1].astype(jnp.int32).reshape(E // 128, 128)

    na, nb = _make_gather(E)(nodes, src, dst)

    eye = jnp.eye(16, dtype=f32)
    a1 = jnp.kron(eye, msg_W1[0:8].astype(f32))
    b1 = jnp.kron(eye, msg_W1[8:16].astype(f32))
    e1 = jnp.kron(eye, msg_W1[16:24].astype(f32))
    bias1 = jnp.tile(msg_b1.astype(f32), 16)[None, :]
    w2a = jnp.kron(eye, msg_W2[:, 0:8].astype(f32))
    w2b = jnp.kron(eye, msg_W2[:, 8:16].astype(f32))
    w2e = jnp.kron(eye, msg_W2[:, 16:24].astype(f32))
    b2a = jnp.tile(msg_b2[0:8].astype(f32), 16)[None, :]
    b2b = jnp.tile(msg_b2[8:16].astype(f32), 16)[None, :]
    b2e = jnp.tile(msg_b2[16:24].astype(f32), 16)[None, :]

    ma2, mb2, ne2 = _make_edge_mlp(E // 16)(
        na.reshape(E // 16, 128), nb.reshape(E // 16, 128),
        edges.reshape(E // 16, 128),
        a1, b1, e1, bias1, w2a, w2b, w2e, b2a, b2b, b2e)

    zeros = jnp.zeros((N_N, CH_N), dtype=f32)
    agg_a, agg_b = _make_scatter(E)(
        src, dst, ma2.reshape(E // 128, 128, CH_N),
        mb2.reshape(E // 128, 128, CH_N), zeros)

    u1n = jnp.kron(eye, upd_W1[0:8].astype(f32))
    u1a = jnp.kron(eye, upd_W1[8:16].astype(f32))
    u1b = jnp.kron(eye, upd_W1[16:24].astype(f32))
    ub1 = jnp.tile(upd_b1.astype(f32), 16)[None, :]
    u2 = jnp.kron(eye, upd_W2.astype(f32))
    ub2 = jnp.tile(upd_b2.astype(f32), 16)[None, :]

    nn2 = _make_node_mlp(N_N // 16)(
        nodes.reshape(N_N // 16, 128),
        agg_a.reshape(N_N // 16, 128), agg_b.reshape(N_N // 16, 128),
        u1n, u1a, u1b, ub1, u2, ub2)

    return nn2.reshape(N_N, CH_N), ne2.reshape(E, CH_N)


# speedup vs baseline: 2.1049x; 2.1049x over previous
import functools

import jax
import jax.numpy as jnp
from jax import lax
from jax.experimental import pallas as pl
from jax.experimental.pallas import tpu as pltpu
from jax.experimental.pallas import tpu_sc as plsc

N_N = 100000
CH_N = 8
CH_E = 8
HID = 16
MAX_VALUE = 1000000.0

NC = 2
NS = 16
NW = NC * NS
G = 20
BLK = G * 128
GNE = 10


def _soft_clamp(x):
    return MAX_VALUE * jnp.tanh(x / MAX_VALUE)


@functools.lru_cache(maxsize=None)
def _make_gather(E):
    nblocks = E // BLK
    mesh = plsc.VectorSubcoreMesh(core_axis_name="c", subcore_axis_name="s")

    @functools.partial(
        pl.kernel,
        mesh=mesh,
        compiler_params=pltpu.CompilerParams(use_tc_tiling_on_sc=False,
                                             needs_layout_passes=False),
        out_type=[jax.ShapeDtypeStruct((E // 128, 128, CH_N), jnp.float32),
                  jax.ShapeDtypeStruct((E // 128, 128, CH_N), jnp.float32),
                  jax.ShapeDtypeStruct((E // 128, CH_N, 128), jnp.float32)],
        scratch_types=[pltpu.VMEM((G, 128), jnp.int32),
                       pltpu.VMEM((G, 128), jnp.int32),
                       pltpu.VMEM((G, 128, CH_N), jnp.float32),
                       pltpu.VMEM((G, 128, CH_N), jnp.float32),
                       pltpu.VMEM((G, CH_N, 128), jnp.float32),
                       pltpu.VMEM((G, CH_N, 128), jnp.float32),
                       pltpu.SemaphoreType.DMA,
                       pltpu.SemaphoreType.DMA,
                       pltpu.SemaphoreType.DMA,
                       pltpu.SemaphoreType.DMA],
    )
    def gather_k(nodes_hbm, src_hbm, dst_hbm, eg_hbm,
                 outa_hbm, outb_hbm, ego_hbm,
                 ia_v, ib_v, ra_v, rb_v, ein_v, eout_v,
                 sema, semb, semld, semw):
        wid = lax.axis_index("s") * NC + lax.axis_index("c")
        nfull = nblocks // NW
        extra = nblocks - nfull * NW
        count = nfull + jnp.where(wid < extra, 1, 0)
        iota = jnp.arange(16, dtype=jnp.int32)
        idiv = iota // 8
        imod = jnp.bitwise_and(iota, 7)

        def in_loads(b):
            return [pltpu.make_async_copy(src_hbm.at[pl.ds(b * G, G)],
                                          ia_v, semld),
                    pltpu.make_async_copy(dst_hbm.at[pl.ds(b * G, G)],
                                          ib_v, semld),
                    pltpu.make_async_copy(eg_hbm.at[pl.ds(b * G, G)],
                                          ein_v, semld)]

        def out_writes(b):
            return [pltpu.make_async_copy(ra_v,
                                          outa_hbm.at[pl.ds(b * G, G)],
                                          semw),
                    pltpu.make_async_copy(rb_v,
                                          outb_hbm.at[pl.ds(b * G, G)],
                                          semw),
                    pltpu.make_async_copy(eout_v,
                                          ego_hbm.at[pl.ds(b * G, G)],
                                          semw)]

        @pl.when(count > 0)
        def _():
            for cp in in_loads(wid):
                cp.start()

        def body(i, carry):
            b = wid + i * NW
            for cp in in_loads(b):
                cp.wait()

            @pl.when(i > 0)
            def _():
                for cp in out_writes(b - NW):
                    cp.wait()

            copies = []
            for j in range(G):
                copies.append(pltpu.async_copy(
                    nodes_hbm.at[ia_v.at[j]], ra_v.at[j], sema))
                copies.append(pltpu.async_copy(
                    nodes_hbm.at[ib_v.at[j]], rb_v.at[j], semb))

            def xpose(j, c2):
                i2, o2 = ein_v.at[j], eout_v.at[j]
                for m in range(8):
                    for q in range(8):
                        va = plsc.load_gather(
                            i2, [imod, 16 * m + 2 * q + idiv])
                        o2[m, pl.ds(16 * q, 16)] = va
                return c2

            lax.fori_loop(0, G, xpose, 0)
            for c in copies:
                c.wait()
            for cp in out_writes(b):
                cp.start()

            @pl.when(i + 1 < count)
            def _():
                for cp in in_loads(b + NW):
                    cp.start()

            return carry

        lax.fori_loop(0, count, body, 0)

        @pl.when(count > 0)
        def _():
            for cp in out_writes(wid + (count - 1) * NW):
                cp.wait()

    return gather_k


@functools.lru_cache(maxsize=None)
def _make_scatter(E):
    nblocks = E // BLK
    share = (N_N // NS) // 8 * 8
    last = N_N - (NS - 1) * share
    mesh = plsc.VectorSubcoreMesh(core_axis_name="c", subcore_axis_name="s")

    @functools.partial(
        pl.kernel,
        mesh=mesh,
        compiler_params=pltpu.CompilerParams(use_tc_tiling_on_sc=False,
                                             needs_layout_passes=False),
        out_type=[jax.ShapeDtypeStruct((N_N, CH_N), jnp.float32),
                  jax.ShapeDtypeStruct((N_N, CH_N), jnp.float32),
                  jax.ShapeDtypeStruct((E // 128, CH_N, 128), jnp.float32)],
        scratch_types=[pltpu.VMEM((2, G, 128), jnp.int32),
                       pltpu.VMEM((2, G, 128, CH_N), jnp.float32),
                       pltpu.VMEM((GNE, CH_N, 128), jnp.float32),
                       pltpu.VMEM((GNE, CH_N, 128), jnp.float32),
                       pltpu.VMEM_SHARED((N_N, CH_N), jnp.float32),
                       pltpu.SemaphoreType.DMA,
                       pltpu.SemaphoreType.DMA,
                       pltpu.SemaphoreType.DMA,
                       pltpu.SemaphoreType.DMA],
    )
    def scatter_k(src_hbm, dst_hbm, ma_hbm, mb_hbm, ne_hbm, zeros_hbm,
                  outa_hbm, outb_hbm, nep_hbm,
                  idx2_v, upd2_v, nin_v, nout_v, agg_sh,
                  sem, semld, semne, semnw):
        cid = lax.axis_index("c")
        sid = lax.axis_index("s")
        wid = sid * NC + cid
        iota = jnp.arange(16, dtype=jnp.int32)

        nblk2 = E // (GNE * 128)
        nfull2 = nblk2 // NW
        extra2 = nblk2 - nfull2 * NW
        count2 = nfull2 + jnp.where(wid < extra2, 1, 0)

        def ne_load(b):
            return pltpu.make_async_copy(ne_hbm.at[pl.ds(b * GNE, GNE)],
                                         nin_v, semne)

        def ne_write(b):
            return pltpu.make_async_copy(nout_v,
                                         nep_hbm.at[pl.ds(b * GNE, GNE)],
                                         semnw)

        @pl.when(count2 > 0)
        def _():
            ne_load(wid).start()

        def ne_body(i, carry):
            b = wid + i * NW
            ne_load(b).wait()

            @pl.when(i > 0)
            def _():
                ne_write(b - NW).wait()

            def xpose(j, c2):
                i2, o2 = nin_v.at[j], nout_v.at[j]
                for c in range(CH_N):
                    for m in range(8):
                        va = plsc.load_gather(i2, [jnp.full((16,), m,
                                                           jnp.int32),
                                                   8 * iota + c])
                        o2[c, pl.ds(16 * m, 16)] = va
                return c2

            lax.fori_loop(0, GNE, xpose, 0)
            ne_write(b).start()

            @pl.when(i + 1 < count2)
            def _():
                ne_load(b + NW).start()

            return carry

        lax.fori_loop(0, count2, ne_body, 0)

        @pl.when(count2 > 0)
        def _():
            ne_write(wid + (count2 - 1) * NW).wait()

        @pl.when(sid < NS - 1)
        def _():
            pltpu.sync_copy(zeros_hbm.at[pl.ds(sid * share, share)],
                            agg_sh.at[pl.ds(sid * share, share)])

        @pl.when(sid == NS - 1)
        def _():
            pltpu.sync_copy(zeros_hbm.at[pl.ds((NS - 1) * share, last)],
                            agg_sh.at[pl.ds((NS - 1) * share, last)])

        plsc.subcore_barrier()

        nfull = nblocks // NS
        extra = nblocks - nfull * NS
        count = nfull + jnp.where(sid < extra, 1, 0)

        def fire_loads(b, s):
            @pl.when(cid == 0)
            def _():
                pltpu.async_copy(src_hbm.at[pl.ds(b * G, G)],
                                 idx2_v.at[s], semld)
                pltpu.async_copy(ma_hbm.at[pl.ds(b * G, G)],
                                 upd2_v.at[s], semld)

            @pl.when(cid == 1)
            def _():
                pltpu.async_copy(dst_hbm.at[pl.ds(b * G, G)],
                                 idx2_v.at[s], semld)
                pltpu.async_copy(mb_hbm.at[pl.ds(b * G, G)],
                                 upd2_v.at[s], semld)

        def wait_loads(b, s):
            pltpu.make_async_copy(src_hbm.at[pl.ds(b * G, G)],
                                  idx2_v.at[s], semld).wait()
            pltpu.make_async_copy(ma_hbm.at[pl.ds(b * G, G)],
                                  upd2_v.at[s], semld).wait()

        def drain_adds(b, s):
            for j in range(G):
                pltpu.make_async_copy(
                    upd2_v.at[s].at[j],
                    agg_sh.at[idx2_v.at[s].at[j]], sem).wait()

        @pl.when(count > 0)
        def _():
            fire_loads(sid, 0)

        def body(i, carry):
            b = sid + i * NS
            s = lax.rem(i, 2)
            wait_loads(b, s)

            @pl.when(i > 0)
            def _():
                drain_adds(b - NS, 1 - s)

            @pl.when(i + 1 < count)
            def _():
                fire_loads(b + NS, 1 - s)

            for j in range(G):
                pltpu.async_copy(upd2_v.at[s].at[j],
                                 agg_sh.at[idx2_v.at[s].at[j]],
                                 sem, add=True)
            return carry

        lax.fori_loop(0, count, body, 0)

        @pl.when(count > 0)
        def _():
            drain_adds(sid + (count - 1) * NS, lax.rem(count - 1, 2))

        plsc.subcore_barrier()

        @pl.when((cid == 0) & (sid < NS - 1))
        def _():
            pltpu.sync_copy(agg_sh.at[pl.ds(sid * share, share)],
                            outa_hbm.at[pl.ds(sid * share, share)])

        @pl.when((cid == 0) & (sid == NS - 1))
        def _():
            pltpu.sync_copy(agg_sh.at[pl.ds((NS - 1) * share, last)],
                            outa_hbm.at[pl.ds((NS - 1) * share, last)])

        @pl.when((cid == 1) & (sid < NS - 1))
        def _():
            pltpu.sync_copy(agg_sh.at[pl.ds(sid * share, share)],
                            outb_hbm.at[pl.ds(sid * share, share)])

        @pl.when((cid == 1) & (sid == NS - 1))
        def _():
            pltpu.sync_copy(agg_sh.at[pl.ds((NS - 1) * share, last)],
                            outb_hbm.at[pl.ds((NS - 1) * share, last)])

    return scatter_k


def _bmm(x, w):
    return jax.lax.dot(x.astype(jnp.bfloat16), w.astype(jnp.bfloat16),
                       preferred_element_type=jnp.float32)


def _edge_body(na_ref, nb_ref, eg_ref, a1_ref, b1_ref, e1_ref, bias1_ref,
               w2a_ref, w2b_ref, w2e_ref, b2a_ref, b2b_ref, b2e_ref,
               ma_ref, mb_ref, ne_ref):
    eg = eg_ref[...]
    h = jnp.tanh(_bmm(na_ref[...], a1_ref[...])
                 + _bmm(nb_ref[...], b1_ref[...])
                 + _bmm(eg, e1_ref[...]) + bias1_ref[...])
    ma_ref[...] = _bmm(h, w2a_ref[...]) + b2a_ref[...]
    mb_ref[...] = _bmm(h, w2b_ref[...]) + b2b_ref[...]
    ne_ref[...] = _soft_clamp(eg + _bmm(h, w2e_ref[...]) + b2e_ref[...])


@functools.lru_cache(maxsize=None)
def _make_edge_mlp(rows):
    R = 1000
    grid = rows // R
    full = lambda shape: pl.BlockSpec(shape, lambda i: (0, 0))
    blk = pl.BlockSpec((R, 128), lambda i: (i, 0))
    return pl.pallas_call(
        _edge_body,
        grid=(grid,),
        in_specs=[blk, blk, blk,
                  full((128, 256)), full((128, 256)), full((128, 256)),
                  full((1, 256)),
                  full((256, 128)), full((256, 128)), full((256, 128)),
                  full((1, 128)), full((1, 128)), full((1, 128))],
        out_specs=[blk, blk, blk],
        out_shape=[jax.ShapeDtypeStruct((rows, 128), jnp.float32)] * 3,
    )


def _node_body(nd_ref, aa_ref, ab_ref, u1n_ref, u1a_ref, u1b_ref, ub1_ref,
               u2_ref, ub2_ref, out_ref):
    nd = nd_ref[...]
    hu = jnp.tanh(nd @ u1n_ref[...] + aa_ref[...] @ u1a_ref[...]
                  + ab_ref[...] @ u1b_ref[...] + ub1_ref[...])
    out_ref[...] = _soft_clamp(nd + hu @ u2_ref[...] + ub2_ref[...])


@functools.lru_cache(maxsize=None)
def _make_node_mlp(rows):
    return pl.pallas_call(
        _node_body,
        out_shape=jax.ShapeDtypeStruct((rows, 128), jnp.float32),
    )


def kernel(nodes, edges, edge_index, msg_W1, msg_b1, msg_W2, msg_b2,
           upd_W1, upd_b1, upd_W2, upd_b2):
    E = edges.shape[0]
    f32 = jnp.float32
    src = edge_index[0].astype(jnp.int32).reshape(E // 128, 128)
    dst = edge_index[1].astype(jnp.int32).reshape(E // 128, 128)

    eg3 = edges.T.reshape(CH_N, E // 128, 128).transpose(1, 0, 2)
    na, nb, ego = _make_gather(E)(nodes, src, dst, eg3)
    eg_int = ego.reshape(E // 16, 128)

    eye = jnp.eye(16, dtype=f32)
    a1 = jnp.kron(eye, msg_W1[0:8].astype(f32))
    b1 = jnp.kron(eye, msg_W1[8:16].astype(f32))
    e1 = jnp.kron(eye, msg_W1[16:24].astype(f32))
    bias1 = jnp.tile(msg_b1.astype(f32), 16)[None, :]
    w2a = jnp.kron(eye, msg_W2[:, 0:8].astype(f32))
    w2b = jnp.kron(eye, msg_W2[:, 8:16].astype(f32))
    w2e = jnp.kron(eye, msg_W2[:, 16:24].astype(f32))
    b2a = jnp.tile(msg_b2[0:8].astype(f32), 16)[None, :]
    b2b = jnp.tile(msg_b2[8:16].astype(f32), 16)[None, :]
    b2e = jnp.tile(msg_b2[16:24].astype(f32), 16)[None, :]

    ma2, mb2, ne2 = _make_edge_mlp(E // 16)(
        na.reshape(E // 16, 128), nb.reshape(E // 16, 128), eg_int,
        a1, b1, e1, bias1, w2a, w2b, w2e, b2a, b2b, b2e)

    zeros = jnp.zeros((N_N, CH_N), dtype=f32)
    agg_a, agg_b, nep3 = _make_scatter(E)(
        src, dst, ma2.reshape(E // 128, 128, CH_N),
        mb2.reshape(E // 128, 128, CH_N),
        ne2.reshape(E // 128, CH_N, 128), zeros)
    new_edges = nep3.transpose(1, 0, 2).reshape(CH_N, E).T

    u1n = jnp.kron(eye, upd_W1[0:8].astype(f32))
    u1a = jnp.kron(eye, upd_W1[8:16].astype(f32))
    u1b = jnp.kron(eye, upd_W1[16:24].astype(f32))
    ub1 = jnp.tile(upd_b1.astype(f32), 16)[None, :]
    u2 = jnp.kron(eye, upd_W2.astype(f32))
    ub2 = jnp.tile(upd_b2.astype(f32), 16)[None, :]

    nn2 = _make_node_mlp(N_N // 16)(
        nodes.reshape(N_N // 16, 128),
        agg_a.reshape(N_N // 16, 128), agg_b.reshape(N_N // 16, 128),
        u1n, u1a, u1b, ub1, u2, ub2)

    return nn2.reshape(N_N, CH_N), new_edges
